# baseline (speedup 1.0000x reference)
import functools

import jax
import jax.numpy as jnp
from jax import lax
from jax.experimental import pallas as pl
from jax.experimental.pallas import tpu as pltpu
from jax.experimental.pallas import tpu_sc as plsc

DIM = 128
NC, NS = 2, 16
NW = NC * NS
CHUNK = 64


def _fold_kernel(emb_ref, w_ref, b_ref, out_ref):
    out_ref[...] = lax.dot_general(
        emb_ref[...], w_ref[...],
        dimension_numbers=(((1,), (1,)), ((), ())),
        preferred_element_type=jnp.float32,
    ) + b_ref[...]


def _fold_table(emb_pad, W, b):
    rows = emb_pad.shape[0]
    return pl.pallas_call(
        _fold_kernel,
        out_shape=jax.ShapeDtypeStruct((rows, DIM), jnp.float32),
    )(emb_pad, W, b.reshape(1, DIM))


@functools.lru_cache(maxsize=None)
def _make_gather(n_total, rows):
    assert n_total % (NW * CHUNK) == 0
    per_w = n_total // NW
    n_chunks = per_w // CHUNK
    assert n_chunks % 2 == 0
    mesh = plsc.VectorSubcoreMesh(
        core_axis_name="c", subcore_axis_name="s",
        num_cores=NC, num_subcores=NS)

    @functools.partial(
        pl.kernel, mesh=mesh,
        out_type=jax.ShapeDtypeStruct((n_total * DIM,), jnp.float32),
        scratch_types=[
            pltpu.VMEM((rows * DIM,), jnp.float32),
            pltpu.VMEM((per_w,), jnp.int32),
            pltpu.VMEM((CHUNK * DIM,), jnp.float32),
            pltpu.VMEM((CHUNK * DIM,), jnp.float32),
            pltpu.SemaphoreType.DMA,
            pltpu.SemaphoreType.DMA,
        ],
        compiler_params=pltpu.CompilerParams(needs_layout_passes=False),
    )
    def gather(table_hbm, idx_hbm, out_hbm, table_v, idx_v, buf_a, buf_b, sa, sb):
        wid = lax.axis_index("s") * NC + lax.axis_index("c")
        base = wid * per_w
        pltpu.sync_copy(table_hbm, table_v)
        pltpu.sync_copy(idx_hbm.at[pl.ds(base, per_w)], idx_v)
        iota = lax.iota(jnp.int32, 16)

        def build(buf, t):
            for g in range(CHUNK // 16):
                idxv = idx_v[pl.ds(t * CHUNK + g * 16, 16)]
                tbase = idxv * DIM
                obase = (iota + g * 16) * DIM
                for c in range(DIM):
                    vals = plsc.load_gather(table_v, [tbase + c])
                    plsc.store_scatter(buf, [obase + c], vals)

        def store_start(buf, t, sem):
            off = (base + t * CHUNK) * DIM
            pltpu.async_copy(buf, out_hbm.at[pl.ds(off, CHUNK * DIM)], sem)

        def drain(buf, sem):
            pltpu.make_async_copy(
                buf, out_hbm.at[pl.ds(base * DIM, CHUNK * DIM)], sem).wait()

        @pl.loop(0, n_chunks // 2)
        def _(i):
            t0 = 2 * i

            @pl.when(i > 0)
            def _():
                drain(buf_a, sa)

            build(buf_a, t0)
            store_start(buf_a, t0, sa)

            @pl.when(i > 0)
            def _():
                drain(buf_b, sb)

            build(buf_b, t0 + 1)
            store_start(buf_b, t0 + 1, sb)

        drain(buf_a, sa)
        drain(buf_b, sb)

    return gather


def kernel(x, embed_table, W, b):
    B, C, H, W_ = x.shape
    L = C * H * W_
    idx = x.reshape(-1).astype(jnp.int32)
    vocab = embed_table.shape[0]
    rows = max(8, -(-vocab // 8) * 8)
    emb_pad = jnp.zeros((rows, DIM), embed_table.dtype).at[:vocab].set(embed_table)
    proj = _fold_table(emb_pad, W, b)
    out = _make_gather(idx.shape[0], rows)(proj.reshape(-1), idx)
    return out.reshape(B, L, DIM)


# speedup vs baseline: 1.0017x; 1.0017x over previous
import functools

import jax
import jax.numpy as jnp
from jax import lax
from jax.experimental import pallas as pl
from jax.experimental.pallas import tpu as pltpu
from jax.experimental.pallas import tpu_sc as plsc

DIM = 128
NC, NS = 2, 16
NW = NC * NS
CHUNK = 256


def _fold_kernel(emb_ref, w_ref, b_ref, out_ref):
    out_ref[...] = lax.dot_general(
        emb_ref[...], w_ref[...],
        dimension_numbers=(((1,), (1,)), ((), ())),
        preferred_element_type=jnp.float32,
    ) + b_ref[...]


def _fold_table(emb_pad, W, b):
    rows = emb_pad.shape[0]
    return pl.pallas_call(
        _fold_kernel,
        out_shape=jax.ShapeDtypeStruct((rows, DIM), jnp.float32),
    )(emb_pad, W, b.reshape(1, DIM))


@functools.lru_cache(maxsize=None)
def _make_gather(n_total, rows):
    assert n_total % (NW * CHUNK) == 0
    per_w = n_total // NW
    n_chunks = per_w // CHUNK
    assert n_chunks % 2 == 0
    mesh = plsc.VectorSubcoreMesh(
        core_axis_name="c", subcore_axis_name="s",
        num_cores=NC, num_subcores=NS)

    @functools.partial(
        pl.kernel, mesh=mesh,
        out_type=jax.ShapeDtypeStruct((n_total * DIM,), jnp.float32),
        scratch_types=[
            pltpu.VMEM((rows * DIM,), jnp.float32),
            pltpu.VMEM((per_w,), jnp.int32),
            pltpu.VMEM((CHUNK * DIM,), jnp.float32),
            pltpu.VMEM((CHUNK * DIM,), jnp.float32),
            pltpu.SemaphoreType.DMA,
            pltpu.SemaphoreType.DMA,
        ],
        compiler_params=pltpu.CompilerParams(needs_layout_passes=False),
    )
    def gather(table_hbm, idx_hbm, out_hbm, table_v, idx_v, buf_a, buf_b, sa, sb):
        wid = lax.axis_index("s") * NC + lax.axis_index("c")
        base = wid * per_w
        pltpu.sync_copy(table_hbm, table_v)
        pltpu.sync_copy(idx_hbm.at[pl.ds(base, per_w)], idx_v)
        iota = lax.iota(jnp.int32, 16)

        def build(buf, t):
            @pl.loop(0, CHUNK // 16)
            def _(g):
                idxv = idx_v[pl.ds(t * CHUNK + g * 16, 16)]
                tbase = idxv * DIM
                obase = (iota + g * 16) * DIM
                for c in range(DIM):
                    vals = plsc.load_gather(table_v, [tbase + c])
                    plsc.store_scatter(buf, [obase + c], vals)

        def store_start(buf, t, sem):
            off = (base + t * CHUNK) * DIM
            pltpu.async_copy(buf, out_hbm.at[pl.ds(off, CHUNK * DIM)], sem)

        def drain(buf, sem):
            pltpu.make_async_copy(
                buf, out_hbm.at[pl.ds(base * DIM, CHUNK * DIM)], sem).wait()

        @pl.loop(0, n_chunks // 2)
        def _(i):
            t0 = 2 * i

            @pl.when(i > 0)
            def _():
                drain(buf_a, sa)

            build(buf_a, t0)
            store_start(buf_a, t0, sa)

            @pl.when(i > 0)
            def _():
                drain(buf_b, sb)

            build(buf_b, t0 + 1)
            store_start(buf_b, t0 + 1, sb)

        drain(buf_a, sa)
        drain(buf_b, sb)

    return gather


def kernel(x, embed_table, W, b):
    B, C, H, W_ = x.shape
    L = C * H * W_
    idx = x.reshape(-1).astype(jnp.int32)
    vocab = embed_table.shape[0]
    rows = max(8, -(-vocab // 8) * 8)
    emb_pad = jnp.zeros((rows, DIM), embed_table.dtype).at[:vocab].set(embed_table)
    proj = _fold_table(emb_pad, W, b)
    out = _make_gather(idx.shape[0], rows)(proj.reshape(-1), idx)
    return out.reshape(B, L, DIM)


# speedup vs baseline: 3.4445x; 3.4386x over previous
import functools

import jax
import jax.numpy as jnp
from jax import lax
from jax.experimental import pallas as pl
from jax.experimental.pallas import tpu as pltpu
from jax.experimental.pallas import tpu_sc as plsc

DIM = 128
NC, NS = 2, 16
NW = NC * NS
PCHUNK = 128
NPAIR = 128


def _make_prep(vocab):
    def prep_kernel(emb_ref, w_ref, b_ref, ea_ref, eb_ref, tab_ref, pid_ref):
        proj = lax.dot_general(
            emb_ref[...], w_ref[...],
            dimension_numbers=(((1,), (1,)), ((), ())),
            preferred_element_type=jnp.float32,
        ) + b_ref[...]
        vocab_rows = emb_ref.shape[0]
        pid = lax.iota(jnp.int32, NPAIR)
        a = jnp.minimum(pid // vocab, vocab_rows - 1)
        b2 = pid % vocab
        sel = lax.iota(jnp.int32, vocab_rows)
        onehot_a = (a[:, None] == sel[None, :]).astype(jnp.float32)
        onehot_b = (b2[:, None] == sel[None, :]).astype(jnp.float32)
        tab_ref[:, :DIM] = jnp.dot(
            onehot_a, proj, preferred_element_type=jnp.float32)
        tab_ref[:, DIM:] = jnp.dot(
            onehot_b, proj, preferred_element_type=jnp.float32)
        pid_ref[...] = ea_ref[...] * vocab + eb_ref[...]

    return prep_kernel


def _prep(emb_pad, W, b, ea, eb, vocab):
    n2 = ea.shape[0]
    tab, pid = pl.pallas_call(
        _make_prep(vocab),
        out_shape=(
            jax.ShapeDtypeStruct((NPAIR, 2 * DIM), jnp.float32),
            jax.ShapeDtypeStruct((n2 // DIM, DIM), jnp.int32),
        ),
    )(emb_pad, W, b.reshape(1, DIM), ea.reshape(n2 // DIM, DIM),
      eb.reshape(n2 // DIM, DIM))
    return tab.reshape(-1), pid.reshape(-1)


@functools.lru_cache(maxsize=None)
def _make_gather(n_pairs):
    assert n_pairs % (NW * PCHUNK) == 0
    per_w = n_pairs // NW
    n_chunks = per_w // PCHUNK
    assert n_chunks % 2 == 0
    row_f = 2 * DIM
    chunk_f = PCHUNK * row_f
    mesh = plsc.VectorSubcoreMesh(
        core_axis_name="c", subcore_axis_name="s",
        num_cores=NC, num_subcores=NS)

    @functools.partial(
        pl.kernel, mesh=mesh,
        out_type=jax.ShapeDtypeStruct((n_pairs * row_f,), jnp.float32),
        scratch_types=[
            pltpu.VMEM((NPAIR * row_f,), jnp.float32),
            pltpu.VMEM((per_w,), jnp.int32),
            pltpu.VMEM((chunk_f,), jnp.float32),
            pltpu.VMEM((chunk_f,), jnp.float32),
            pltpu.SemaphoreType.DMA,
            pltpu.SemaphoreType.DMA,
        ],
        compiler_params=pltpu.CompilerParams(needs_layout_passes=False),
    )
    def gather(tab_hbm, pid_hbm, out_hbm, tab_v, pid_v, buf_a, buf_b, sa, sb):
        wid = lax.axis_index("s") * NC + lax.axis_index("c")
        base = wid * per_w
        pltpu.sync_copy(tab_hbm, tab_v)
        pltpu.sync_copy(pid_hbm.at[pl.ds(base, per_w)], pid_v)

        def build(buf, t):
            @plsc.parallel_loop(0, PCHUNK // 16, unroll=2)
            def _(g):
                pairbase = g * 16
                idxv = pid_v[pl.ds(t * PCHUNK + pairbase, 16)]
                for j in range(16):
                    src = idxv[j] * row_f
                    dst = (pairbase + j) * row_f
                    for c in range(row_f // 16):
                        buf[pl.ds(dst + c * 16, 16)] = (
                            tab_v[pl.ds(src + c * 16, 16)])

        def store_start(buf, t, sem):
            off = (base + t * PCHUNK) * row_f
            pltpu.async_copy(buf, out_hbm.at[pl.ds(off, chunk_f)], sem)

        def drain(buf, sem):
            pltpu.make_async_copy(
                buf, out_hbm.at[pl.ds(base * row_f, chunk_f)], sem).wait()

        @pl.loop(0, n_chunks // 2)
        def _(i):
            t0 = 2 * i

            @pl.when(i > 0)
            def _():
                drain(buf_a, sa)

            build(buf_a, t0)
            store_start(buf_a, t0, sa)

            @pl.when(i > 0)
            def _():
                drain(buf_b, sb)

            build(buf_b, t0 + 1)
            store_start(buf_b, t0 + 1, sb)

        drain(buf_a, sa)
        drain(buf_b, sb)

    return gather


def kernel(x, embed_table, W, b):
    B, C, H, W_ = x.shape
    L = C * H * W_
    idx = x.reshape(-1).astype(jnp.int32)
    vocab = embed_table.shape[0]
    assert vocab * vocab <= NPAIR
    rows = max(8, -(-vocab // 8) * 8)
    emb_pad = jnp.zeros((rows, DIM), embed_table.dtype).at[:vocab].set(embed_table)
    pairs = idx.reshape(-1, 2)
    tab, pid = _prep(emb_pad, W, b, pairs[:, 0], pairs[:, 1], vocab)
    out = _make_gather(pid.shape[0])(tab, pid)
    return out.reshape(B, L, DIM)


# speedup vs baseline: 6.9044x; 2.0044x over previous
import functools

import jax
import jax.numpy as jnp
from jax import lax
from jax.experimental import pallas as pl
from jax.experimental.pallas import tpu as pltpu
from jax.experimental.pallas import tpu_sc as plsc

DIM = 128
NC, NS = 2, 16
NW = NC * NS
CHUNK = 256


def _fold_kernel(emb_ref, w_ref, b_ref, out_ref):
    out_ref[...] = lax.dot_general(
        emb_ref[...], w_ref[...],
        dimension_numbers=(((1,), (1,)), ((), ())),
        preferred_element_type=jnp.float32,
    ) + b_ref[...]


def _fold_table(emb_pad, W, b):
    rows = emb_pad.shape[0]
    return pl.pallas_call(
        _fold_kernel,
        out_shape=jax.ShapeDtypeStruct((rows, DIM), jnp.float32),
    )(emb_pad, W, b.reshape(1, DIM))


@functools.lru_cache(maxsize=None)
def _make_gather(n_total, rows):
    assert n_total % (NW * CHUNK) == 0
    per_w = n_total // NW
    n_chunks = per_w // CHUNK
    assert n_chunks % 2 == 0
    mesh = plsc.VectorSubcoreMesh(
        core_axis_name="c", subcore_axis_name="s",
        num_cores=NC, num_subcores=NS)

    @functools.partial(
        pl.kernel, mesh=mesh,
        out_type=jax.ShapeDtypeStruct((n_total * DIM,), jnp.float32),
        scratch_types=[
            pltpu.VMEM((rows * DIM,), jnp.float32),
            pltpu.VMEM((per_w,), jnp.int32),
            pltpu.VMEM((CHUNK * DIM,), jnp.float32),
            pltpu.VMEM((CHUNK * DIM,), jnp.float32),
            pltpu.SemaphoreType.DMA,
            pltpu.SemaphoreType.DMA,
        ],
        compiler_params=pltpu.CompilerParams(needs_layout_passes=False),
    )
    def gather(table_hbm, idx_hbm, out_hbm, table_v, idx_v, buf_a, buf_b, sa, sb):
        wid = lax.axis_index("s") * NC + lax.axis_index("c")
        base = wid * per_w
        pltpu.sync_copy(table_hbm, table_v)
        pltpu.sync_copy(idx_hbm.at[pl.ds(base, per_w)], idx_v)
        iota = lax.iota(jnp.int32, 16)

        def build(buf, t):
            @plsc.parallel_loop(0, CHUNK // 16, unroll=3)
            def _(g):
                rowbase = g * 16
                idxv = idx_v[pl.ds(t * CHUNK + rowbase, 16)]
                for j in range(16):
                    r = idxv[j]
                    src = r * DIM
                    dst = (rowbase + j) * DIM
                    for c in range(DIM // 16):
                        buf[pl.ds(dst + c * 16, 16)] = (
                            table_v[pl.ds(src + c * 16, 16)])

        def store_start(buf, t, sem):
            off = (base + t * CHUNK) * DIM
            pltpu.async_copy(buf, out_hbm.at[pl.ds(off, CHUNK * DIM)], sem)

        def drain(buf, sem):
            pltpu.make_async_copy(
                buf, out_hbm.at[pl.ds(base * DIM, CHUNK * DIM)], sem).wait()

        @pl.loop(0, n_chunks // 2)
        def _(i):
            t0 = 2 * i

            @pl.when(i > 0)
            def _():
                drain(buf_a, sa)

            build(buf_a, t0)
            store_start(buf_a, t0, sa)

            @pl.when(i > 0)
            def _():
                drain(buf_b, sb)

            build(buf_b, t0 + 1)
            store_start(buf_b, t0 + 1, sb)

        drain(buf_a, sa)
        drain(buf_b, sb)

    return gather


def kernel(x, embed_table, W, b):
    B, C, H, W_ = x.shape
    L = C * H * W_
    idx = x.reshape(-1).astype(jnp.int32)
    vocab = embed_table.shape[0]
    rows = max(8, -(-vocab // 8) * 8)
    emb_pad = jnp.zeros((rows, DIM), embed_table.dtype).at[:vocab].set(embed_table)
    proj = _fold_table(emb_pad, W, b)
    out = _make_gather(idx.shape[0], rows)(proj.reshape(-1), idx)
    return out.reshape(B, L, DIM)


# speedup vs baseline: 7.1109x; 1.0299x over previous
import functools

import jax
import jax.numpy as jnp
from jax import lax
from jax.experimental import pallas as pl
from jax.experimental.pallas import tpu as pltpu
from jax.experimental.pallas import tpu_sc as plsc

DIM = 128
NC, NS = 2, 16
NW = NC * NS
CHUNK = 256
REGION = CHUNK + 16


def _fold_kernel(emb_ref, w_ref, b_ref, out_ref):
    out_ref[...] = lax.dot_general(
        emb_ref[...], w_ref[...],
        dimension_numbers=(((1,), (1,)), ((), ())),
        preferred_element_type=jnp.float32,
    ) + b_ref[...]


def _fold_table(emb_pad, W, b):
    rows = emb_pad.shape[0]
    return pl.pallas_call(
        _fold_kernel,
        out_shape=jax.ShapeDtypeStruct((rows, DIM), jnp.float32),
    )(emb_pad, W, b.reshape(1, DIM))


@functools.lru_cache(maxsize=None)
def _make_gather(n_total, rows, vocab):
    assert n_total % (NW * CHUNK) == 0
    per_w = n_total // NW
    n_chunks = per_w // CHUNK
    assert n_chunks % 2 == 0
    buf_f = (CHUNK + 1) * DIM
    chunk_f = CHUNK * DIM
    mesh = plsc.VectorSubcoreMesh(
        core_axis_name="c", subcore_axis_name="s",
        num_cores=NC, num_subcores=NS)

    @functools.partial(
        pl.kernel, mesh=mesh,
        out_type=jax.ShapeDtypeStruct((n_total * DIM,), jnp.float32),
        scratch_types=[
            pltpu.VMEM((rows * DIM,), jnp.float32),
            pltpu.VMEM((per_w,), jnp.int32),
            pltpu.VMEM((vocab * REGION,), jnp.int32),
            pltpu.VMEM((buf_f,), jnp.float32),
            pltpu.VMEM((buf_f,), jnp.float32),
            pltpu.SemaphoreType.DMA,
            pltpu.SemaphoreType.DMA,
        ],
        compiler_params=pltpu.CompilerParams(needs_layout_passes=False),
    )
    def gather(table_hbm, idx_hbm, out_hbm, table_v, idx_v, pos_v,
               buf_a, buf_b, sa, sb):
        wid = lax.axis_index("s") * NC + lax.axis_index("c")
        base = wid * per_w
        pltpu.sync_copy(table_hbm, table_v)
        pltpu.sync_copy(idx_hbm.at[pl.ds(base, per_w)], idx_v)
        iota = lax.iota(jnp.int32, 16)
        dummy = jnp.full((16,), CHUNK, jnp.int32)

        def build(buf, t):
            @pl.loop(0, CHUNK // 16,
                     init_carry=tuple(jnp.int32(0) for _ in range(vocab)))
            def cnts(g, cnts):
                idxv = idx_v[pl.ds(t * CHUNK + g * 16, 16)]
                posv = iota + g * 16
                new = []
                for v in range(vocab):
                    m = idxv == v
                    plsc.store_compressed(
                        pos_v.at[pl.ds(v * REGION + cnts[v], 16)], posv,
                        mask=m)
                    new.append(cnts[v]
                               + plsc.all_reduce_population_count(m)[0])
                return tuple(new)

            for v in range(vocab):
                pos_v[pl.ds(v * REGION + cnts[v], 16)] = dummy

            for v in range(vocab):
                rowregs = [table_v[pl.ds(v * DIM + c * 16, 16)]
                           for c in range(DIM // 16)]
                n4 = (cnts[v] + 3) >> 2

                @plsc.parallel_loop(0, n4, unroll=2)
                def _(k, v=v, rowregs=rowregs):
                    pw = pos_v[pl.ds(v * REGION + k * 4, 16)]
                    for j in range(4):
                        dst = pw[j] * DIM
                        for c in range(DIM // 16):
                            buf[pl.ds(dst + c * 16, 16)] = rowregs[c]

        def store_start(buf, t, sem):
            off = (base + t * CHUNK) * DIM
            pltpu.async_copy(
                buf.at[pl.ds(0, chunk_f)], out_hbm.at[pl.ds(off, chunk_f)],
                sem)

        def drain(buf, sem):
            pltpu.make_async_copy(
                buf.at[pl.ds(0, chunk_f)],
                out_hbm.at[pl.ds(base * DIM, chunk_f)], sem).wait()

        @pl.loop(0, n_chunks // 2)
        def _(i):
            t0 = 2 * i

            @pl.when(i > 0)
            def _():
                drain(buf_a, sa)

            build(buf_a, t0)
            store_start(buf_a, t0, sa)

            @pl.when(i > 0)
            def _():
                drain(buf_b, sb)

            build(buf_b, t0 + 1)
            store_start(buf_b, t0 + 1, sb)

        drain(buf_a, sa)
        drain(buf_b, sb)

    return gather


def kernel(x, embed_table, W, b):
    B, C, H, W_ = x.shape
    L = C * H * W_
    idx = x.reshape(-1).astype(jnp.int32)
    vocab = embed_table.shape[0]
    rows = max(8, -(-vocab // 8) * 8)
    emb_pad = jnp.zeros((rows, DIM), embed_table.dtype).at[:vocab].set(embed_table)
    proj = _fold_table(emb_pad, W, b)
    out = _make_gather(idx.shape[0], rows, vocab)(proj.reshape(-1), idx)
    return out.reshape(B, L, DIM)


# speedup vs baseline: 7.4275x; 1.0445x over previous
import functools

import jax
import jax.numpy as jnp
from jax import lax
from jax.experimental import pallas as pl
from jax.experimental.pallas import tpu as pltpu
from jax.experimental.pallas import tpu_sc as plsc

DIM = 128
NC, NS = 2, 16
NW = NC * NS
CHUNK = 256
REGION = CHUNK + 16


def _fold_kernel(emb_ref, w_ref, b_ref, out_ref):
    out_ref[...] = lax.dot_general(
        emb_ref[...], w_ref[...],
        dimension_numbers=(((1,), (1,)), ((), ())),
        preferred_element_type=jnp.float32,
    ) + b_ref[...]


def _fold_table(emb_pad, W, b):
    rows = emb_pad.shape[0]
    return pl.pallas_call(
        _fold_kernel,
        out_shape=jax.ShapeDtypeStruct((rows, DIM), jnp.float32),
    )(emb_pad, W, b.reshape(1, DIM))


@functools.lru_cache(maxsize=None)
def _make_gather(n_total, rows, vocab):
    assert n_total % (NW * CHUNK) == 0
    per_w = n_total // NW
    n_chunks = per_w // CHUNK
    assert n_chunks % 2 == 0
    buf_f = (CHUNK + 1) * DIM
    chunk_f = CHUNK * DIM
    mesh = plsc.VectorSubcoreMesh(
        core_axis_name="c", subcore_axis_name="s",
        num_cores=NC, num_subcores=NS)

    @functools.partial(
        pl.kernel, mesh=mesh,
        out_type=jax.ShapeDtypeStruct((n_total * DIM,), jnp.float32),
        scratch_types=[
            pltpu.VMEM((rows * DIM,), jnp.float32),
            pltpu.VMEM((per_w,), jnp.int32),
            pltpu.VMEM((vocab * REGION,), jnp.int32),
            pltpu.VMEM((16,), jnp.int32),
            pltpu.VMEM((buf_f,), jnp.float32),
            pltpu.VMEM((buf_f,), jnp.float32),
            pltpu.SemaphoreType.DMA,
            pltpu.SemaphoreType.DMA,
        ],
        compiler_params=pltpu.CompilerParams(needs_layout_passes=False),
    )
    def gather(table_hbm, idx_hbm, out_hbm, table_v, idx_v, pos_v, cnt_v,
               buf_a, buf_b, sa, sb):
        wid = lax.axis_index("s") * NC + lax.axis_index("c")
        base = wid * per_w
        pltpu.sync_copy(table_hbm, table_v)
        pltpu.sync_copy(idx_hbm.at[pl.ds(base, per_w)], idx_v)
        iota = lax.iota(jnp.int32, 16)
        dummy = jnp.full((16,), CHUNK, jnp.int32)
        ones = jnp.full((16,), 1, jnp.int32)
        zeros = jnp.zeros((16,), jnp.int32)

        def build(buf, t):
            cnt_v[pl.ds(0, 16)] = zeros

            @pl.loop(0, CHUNK // 16)
            def _(g):
                idxv = idx_v[pl.ds(t * CHUNK + g * 16, 16)]
                posv = iota + g * 16
                rank, _last = plsc.scan_count(idxv)
                basev = plsc.load_gather(cnt_v, [idxv])
                dst = idxv * REGION + basev + (rank - 1)
                plsc.store_scatter(pos_v, [dst], posv)
                plsc.addupdate_scatter(cnt_v, [idxv], ones)

            cntv = cnt_v[pl.ds(0, 16)]
            for v in range(vocab):
                pos_v[pl.ds(v * REGION + cntv[v], 16)] = dummy

            for v in range(vocab):
                rowregs = [table_v[pl.ds(v * DIM + c * 16, 16)]
                           for c in range(DIM // 16)]
                n4 = (cntv[v] + 3) >> 2

                @plsc.parallel_loop(0, n4, unroll=2)
                def _(k, v=v, rowregs=rowregs):
                    pw = pos_v[pl.ds(v * REGION + k * 4, 16)]
                    for j in range(4):
                        dst = pw[j] * DIM
                        for c in range(DIM // 16):
                            buf[pl.ds(dst + c * 16, 16)] = rowregs[c]

        def store_start(buf, t, sem):
            off = (base + t * CHUNK) * DIM
            pltpu.async_copy(
                buf.at[pl.ds(0, chunk_f)], out_hbm.at[pl.ds(off, chunk_f)],
                sem)

        def drain(buf, sem):
            pltpu.make_async_copy(
                buf.at[pl.ds(0, chunk_f)],
                out_hbm.at[pl.ds(base * DIM, chunk_f)], sem).wait()

        @pl.loop(0, n_chunks // 2)
        def _(i):
            t0 = 2 * i

            @pl.when(i > 0)
            def _():
                drain(buf_a, sa)

            build(buf_a, t0)
            store_start(buf_a, t0, sa)

            @pl.when(i > 0)
            def _():
                drain(buf_b, sb)

            build(buf_b, t0 + 1)
            store_start(buf_b, t0 + 1, sb)

        drain(buf_a, sa)
        drain(buf_b, sb)

    return gather


def kernel(x, embed_table, W, b):
    B, C, H, W_ = x.shape
    L = C * H * W_
    idx = x.reshape(-1).astype(jnp.int32)
    vocab = embed_table.shape[0]
    rows = max(8, -(-vocab // 8) * 8)
    emb_pad = jnp.zeros((rows, DIM), embed_table.dtype).at[:vocab].set(embed_table)
    proj = _fold_table(emb_pad, W, b)
    out = _make_gather(idx.shape[0], rows, vocab)(proj.reshape(-1), idx)
    return out.reshape(B, L, DIM)


# speedup vs baseline: 8.6934x; 1.1704x over previous
import functools

import jax
import jax.numpy as jnp
from jax import lax
from jax.experimental import pallas as pl
from jax.experimental.pallas import tpu as pltpu
from jax.experimental.pallas import tpu_sc as plsc

DIM = 128
NC, NS = 2, 16
NW = NC * NS
CHUNK = 256
REGION = CHUNK + 16


def _fold_kernel(emb_ref, w_ref, b_ref, out_ref):
    out_ref[...] = lax.dot_general(
        emb_ref[...], w_ref[...],
        dimension_numbers=(((1,), (1,)), ((), ())),
        preferred_element_type=jnp.float32,
    ) + b_ref[...]


def _fold_table(emb_pad, W, b):
    rows = emb_pad.shape[0]
    return pl.pallas_call(
        _fold_kernel,
        out_shape=jax.ShapeDtypeStruct((rows, DIM), jnp.float32),
    )(emb_pad, W, b.reshape(1, DIM))


@functools.lru_cache(maxsize=None)
def _make_gather(n_total, rows, vocab):
    assert n_total % (NW * CHUNK) == 0
    per_w = n_total // NW
    n_chunks = per_w // CHUNK
    assert n_chunks % 2 == 0
    buf_f = (CHUNK + 1) * DIM
    chunk_f = CHUNK * DIM
    mesh = plsc.VectorSubcoreMesh(
        core_axis_name="c", subcore_axis_name="s",
        num_cores=NC, num_subcores=NS)

    @functools.partial(
        pl.kernel, mesh=mesh,
        out_type=jax.ShapeDtypeStruct((n_total * DIM,), jnp.float32),
        scratch_types=[
            pltpu.VMEM((rows * DIM,), jnp.float32),
            pltpu.VMEM((per_w,), jnp.int32),
            pltpu.VMEM((vocab * REGION,), jnp.int32),
            pltpu.VMEM((16,), jnp.int32),
            pltpu.VMEM((buf_f,), jnp.float32),
            pltpu.VMEM((buf_f,), jnp.float32),
            pltpu.SemaphoreType.DMA,
            pltpu.SemaphoreType.DMA,
        ],
        compiler_params=pltpu.CompilerParams(needs_layout_passes=False),
    )
    def gather(table_hbm, idx_hbm, out_hbm, table_v, idx_v, pos_v, cnt_v,
               buf_a, buf_b, sa, sb):
        wid = lax.axis_index("s") * NC + lax.axis_index("c")
        base = wid * per_w
        pltpu.sync_copy(table_hbm, table_v)
        pltpu.sync_copy(idx_hbm.at[pl.ds(base, per_w)], idx_v)
        iota = lax.iota(jnp.int32, 16)
        dummy = jnp.full((16,), CHUNK, jnp.int32)
        ones = jnp.full((16,), 1, jnp.int32)
        zeros = jnp.zeros((16,), jnp.int32)

        def build(buf, t):
            cnt_v[pl.ds(0, 16)] = zeros

            @pl.loop(0, CHUNK // 16)
            def _(g):
                idxv = idx_v[pl.ds(t * CHUNK + g * 16, 16)]
                posv = iota + g * 16
                rank, _last = plsc.scan_count(idxv)
                basev = plsc.load_gather(cnt_v, [idxv])
                dst = idxv * REGION + basev + (rank - 1)
                plsc.store_scatter(pos_v, [dst], posv)
                plsc.addupdate_scatter(cnt_v, [idxv], ones)

            cntv = cnt_v[pl.ds(0, 16)]
            for v in range(vocab):
                pos_v[pl.ds(v * REGION + cntv[v], 16)] = dummy

            for v in range(vocab):
                rowregs = [table_v[pl.ds(v * DIM + c * 16, 16)]
                           for c in range(DIM // 16)]
                n8 = (cntv[v] + 7) >> 3

                @plsc.parallel_loop(0, n8)
                def _(k, v=v, rowregs=rowregs):
                    pw = pos_v[pl.ds(v * REGION + k * 8, 16)]
                    for j in range(8):
                        dst = pw[j] * DIM
                        for c in range(DIM // 16):
                            buf[pl.ds(dst + c * 16, 16)] = rowregs[c]

        def store_start(buf, t, sem):
            off = (base + t * CHUNK) * DIM
            pltpu.async_copy(
                buf.at[pl.ds(0, chunk_f)], out_hbm.at[pl.ds(off, chunk_f)],
                sem)

        def drain(buf, sem):
            pltpu.make_async_copy(
                buf.at[pl.ds(0, chunk_f)],
                out_hbm.at[pl.ds(base * DIM, chunk_f)], sem).wait()

        @pl.loop(0, n_chunks // 2)
        def _(i):
            t0 = 2 * i

            @pl.when(i > 0)
            def _():
                drain(buf_a, sa)

            build(buf_a, t0)
            store_start(buf_a, t0, sa)

            @pl.when(i > 0)
            def _():
                drain(buf_b, sb)

            build(buf_b, t0 + 1)
            store_start(buf_b, t0 + 1, sb)

        drain(buf_a, sa)
        drain(buf_b, sb)

    return gather


def kernel(x, embed_table, W, b):
    B, C, H, W_ = x.shape
    L = C * H * W_
    idx = x.reshape(-1).astype(jnp.int32)
    vocab = embed_table.shape[0]
    rows = max(8, -(-vocab // 8) * 8)
    emb_pad = jnp.zeros((rows, DIM), embed_table.dtype).at[:vocab].set(embed_table)
    proj = _fold_table(emb_pad, W, b)
    out = _make_gather(idx.shape[0], rows, vocab)(proj.reshape(-1), idx)
    return out.reshape(B, L, DIM)
